# Initial kernel scaffold; baseline (speedup 1.0000x reference)
import jax
import jax.numpy as jnp
from jax.experimental import pallas as pl


def kernel(inputs, embedding):
    raise NotImplementedError("write your pallas kernel here")


# speedup vs baseline: 1.3245x; 1.3245x over previous
import jax
import jax.numpy as jnp
from jax.experimental import pallas as pl
from jax.experimental.pallas import tpu as pltpu


def _gather_mm_kernel(idx_ref, hi_ref, lo_ref, out_ref):
    t_pad = hi_ref.shape[0]
    blk = idx_ref.shape[1]
    idx_row = idx_ref[0:1, :]
    iota_t = jax.lax.broadcasted_iota(jnp.int32, (t_pad, blk), 0)
    oh = (iota_t == idx_row).astype(jnp.bfloat16)
    acc = jax.lax.dot_general(
        oh, hi_ref[...], (((0,), (0,)), ((), ())),
        preferred_element_type=jnp.float32)
    acc = acc + jax.lax.dot_general(
        oh, lo_ref[...], (((0,), (0,)), ((), ())),
        preferred_element_type=jnp.float32)
    out_ref[...] = acc


def _positional_gather(inputs, embedding):
    t, d = embedding.shape

    t_pad = ((max(t, 128) + 127) // 128) * 128
    hi = embedding.astype(jnp.bfloat16)
    lo = (embedding - hi.astype(jnp.float32)).astype(jnp.bfloat16)
    hi = jnp.pad(hi, ((0, t_pad - t), (0, 0)))
    lo = jnp.pad(lo, ((0, t_pad - t), (0, 0)))

    idx_flat = inputs.reshape(-1).astype(jnp.int32)
    idx_flat = jnp.clip(idx_flat, 0, t - 1)
    n = idx_flat.shape[0]
    n_pad = ((n + 127) // 128) * 128
    idx_pad = jnp.pad(idx_flat, (0, n_pad - n))

    block = 128
    for cand in (4096, 2048, 1024, 512, 256, 128):
        if n_pad % cand == 0:
            block = cand
            break
    n_blocks = n_pad // block

    idx3 = idx_pad.reshape(n_blocks, 1, block)

    out_flat = pl.pallas_call(
        _gather_mm_kernel,
        out_shape=jax.ShapeDtypeStruct((n_pad, d), jnp.float32),
        grid_spec=pltpu.PrefetchScalarGridSpec(
            num_scalar_prefetch=0,
            grid=(n_blocks,),
            in_specs=[
                pl.BlockSpec((None, 1, block), lambda i: (i, 0, 0)),
                pl.BlockSpec((t_pad, d), lambda i: (0, 0)),
                pl.BlockSpec((t_pad, d), lambda i: (0, 0)),
            ],
            out_specs=pl.BlockSpec((block, d), lambda i: (i, 0)),
        ),
        compiler_params=pltpu.CompilerParams(
            dimension_semantics=("parallel",)),
        cost_estimate=pl.CostEstimate(
            flops=2 * 2 * n_pad * t_pad * d,
            transcendentals=0,
            bytes_accessed=n_pad * 4 + n_pad * d * 4 + 2 * t_pad * d * 2),
    )(idx3, hi, lo)

    if n != n_pad:
        out_flat = out_flat[:n]
    return out_flat.reshape(inputs.shape + (d,))


def kernel(inputs, embedding):
    return _positional_gather(inputs, embedding)
